# baseline (speedup 1.0000x reference)
import functools

import jax
import jax.numpy as jnp
from jax.experimental import pallas as pl
from jax.experimental.pallas import tpu as pltpu
from jax.experimental.pallas import tpu_sc as plsc

E = 8
TOPK = 2
BM = 256
SC_W = 128
SC_C = 64

_VECTOR_MESH = plsc.VectorSubcoreMesh(
    core_axis_name="core", subcore_axis_name="subcore")


def _routing(expert_idxs):
    S = expert_idxs.shape[0] * expert_idxs.shape[1]
    NB = S // BM + E
    flat_e = expert_idxs.reshape(S).astype(jnp.int32)
    onehot = (flat_e[:, None] == jnp.arange(E, dtype=jnp.int32)[None, :])
    onehot_i = onehot.astype(jnp.int32)
    ranks = jnp.cumsum(onehot_i, axis=0) - onehot_i
    counts = jnp.sum(onehot_i, axis=0)
    padded = ((counts + BM - 1) // BM) * BM
    pad_off = jnp.cumsum(padded) - padded
    pos = jnp.sum(jnp.where(onehot, ranks + pad_off[None, :], 0), axis=1)
    pos = pos.astype(jnp.int32)
    bstart = (jnp.arange(NB, dtype=jnp.int32) * BM)[:, None]
    block_expert = jnp.sum((bstart >= pad_off[None, :]).astype(jnp.int32),
                           axis=1) - 1
    block_valid = (bstart[:, 0] < jnp.sum(padded)).astype(jnp.int32)
    return pos, block_expert.astype(jnp.int32), block_valid


@functools.partial(jax.jit, static_argnames=("n_rows",))
def _sc_dispatch(x, tok_idx, pos, n_rows):
    nw = tok_idx.shape[0]
    D = x.shape[1]

    @functools.partial(
        pl.kernel,
        out_type=jax.ShapeDtypeStruct((n_rows, D), x.dtype),
        mesh=_VECTOR_MESH,
        scratch_types=[pltpu.VMEM((SC_C, D), x.dtype)],
    )
    def run(x_hbm, tok_hbm, pos_hbm, o_hbm, buf):
        def body(tok_vmem, pos_vmem):
            for c in range(0, SC_W, SC_C):
                pltpu.sync_copy(x_hbm.at[tok_vmem.at[0, pl.ds(c, SC_C)]], buf)
                pltpu.sync_copy(buf, o_hbm.at[pos_vmem.at[0, pl.ds(c, SC_C)]])

        pltpu.emit_pipeline(
            body,
            grid=(nw,),
            in_specs=[
                pl.BlockSpec((1, SC_W), index_map=lambda i: (i, 0)),
                pl.BlockSpec((1, SC_W), index_map=lambda i: (i, 0)),
            ],
            out_specs=[],
            core_axis_name=("core", "subcore"),
            dimension_semantics=(pltpu.PARALLEL,),
        )(tok_hbm, pos_hbm)

    return run(x, tok_idx, pos)


@jax.jit
def _sc_combine_gather(y_sorted, pos_a, pos_b, dst):
    nw, _ = pos_a.shape
    T = nw * SC_W
    D = y_sorted.shape[1]
    otype = jax.ShapeDtypeStruct((T, D), y_sorted.dtype)

    @functools.partial(
        pl.kernel, out_type=(otype, otype), mesh=_VECTOR_MESH,
        scratch_types=[pltpu.VMEM((SC_C, D), y_sorted.dtype)],
    )
    def run(ys_hbm, pa_hbm, pb_hbm, dst_hbm, oa_hbm, ob_hbm, buf):
        def body(pa_vmem, pb_vmem, dst_vmem):
            for c in range(0, SC_W, SC_C):
                dsts = dst_vmem.at[0, pl.ds(c, SC_C)]
                pltpu.sync_copy(ys_hbm.at[pa_vmem.at[0, pl.ds(c, SC_C)]], buf)
                pltpu.sync_copy(buf, oa_hbm.at[dsts])
                pltpu.sync_copy(ys_hbm.at[pb_vmem.at[0, pl.ds(c, SC_C)]], buf)
                pltpu.sync_copy(buf, ob_hbm.at[dsts])

        pltpu.emit_pipeline(
            body,
            grid=(nw,),
            in_specs=[
                pl.BlockSpec((1, SC_W), index_map=lambda i: (i, 0)),
                pl.BlockSpec((1, SC_W), index_map=lambda i: (i, 0)),
                pl.BlockSpec((1, SC_W), index_map=lambda i: (i, 0)),
            ],
            out_specs=[],
            core_axis_name=("core", "subcore"),
            dimension_semantics=(pltpu.PARALLEL,),
        )(pa_hbm, pb_hbm, dst_hbm)

    return run(y_sorted, pos_a, pos_b, dst)


def _ffn_block(be_ref, bv_ref, x_ref, w1_ref, w2_ref, o_ref):
    @pl.when(bv_ref[pl.program_id(0)] != 0)
    def _():
        half = BM // 2
        for r in range(0, BM, half):
            x = x_ref[pl.ds(r, half), :].astype(jnp.bfloat16)
            h = jnp.dot(x, w1_ref[0], preferred_element_type=jnp.float32)
            h = jax.nn.gelu(h.astype(jnp.bfloat16))
            o_ref[pl.ds(r, half), :] = jnp.dot(
                h, w2_ref[0].astype(jnp.bfloat16),
                preferred_element_type=jnp.float32)


@functools.partial(jax.jit, static_argnames=("nb",))
def _grouped_ffn(xin, W1b, W2, block_expert, block_valid, nb):
    D = xin.shape[1]
    H = W1b.shape[2]
    grid_spec = pltpu.PrefetchScalarGridSpec(
        num_scalar_prefetch=2,
        grid=(nb,),
        in_specs=[
            pl.BlockSpec((BM, D), lambda b, be, bv: (b, 0)),
            pl.BlockSpec((1, D, H), lambda b, be, bv: (be[b], 0, 0)),
            pl.BlockSpec((1, H, D), lambda b, be, bv: (be[b], 0, 0)),
        ],
        out_specs=pl.BlockSpec((BM, D), lambda b, be, bv: (b, 0)),
    )
    return pl.pallas_call(
        _ffn_block,
        grid_spec=grid_spec,
        out_shape=jax.ShapeDtypeStruct((nb * BM, D), jnp.float32),
    )(block_expert, block_valid, xin, W1b, W2)


def _combine_block(ya_ref, yb_ref, pa_ref, pb_ref, o_ref):
    o_ref[...] = pa_ref[...] * ya_ref[...] + pb_ref[...] * yb_ref[...]


@jax.jit
def _weighted_combine(ya, yb, pa, pb):
    T, D = ya.shape
    BT = 512
    return pl.pallas_call(
        _combine_block,
        grid=(T // BT,),
        in_specs=[
            pl.BlockSpec((BT, D), lambda i: (i, 0)),
            pl.BlockSpec((BT, D), lambda i: (i, 0)),
            pl.BlockSpec((BT, 1), lambda i: (i, 0)),
            pl.BlockSpec((BT, 1), lambda i: (i, 0)),
        ],
        out_specs=pl.BlockSpec((BT, D), lambda i: (i, 0)),
        out_shape=jax.ShapeDtypeStruct((T, D), jnp.float32),
    )(ya, yb, pa, pb)


def kernel(x, expert_p, expert_idxs, W1, W2):
    T, D = x.shape
    S = T * TOPK
    NB = S // BM + E

    pos, block_expert, block_valid = _routing(expert_idxs)


    tok_idx = (jnp.arange(S, dtype=jnp.int32) // TOPK).reshape(-1, SC_W)
    xin = _sc_dispatch(x, tok_idx, pos.reshape(-1, SC_W), NB * BM)

    W1b = W1.astype(jnp.bfloat16)
    y_sorted = _grouped_ffn(xin, W1b, W2, block_expert, block_valid, NB)

    pos2 = pos.reshape(T, TOPK)
    dst = jnp.arange(T, dtype=jnp.int32).reshape(-1, SC_W)
    ya, yb = _sc_combine_gather(
        y_sorted,
        pos2[:, 0].reshape(-1, SC_W),
        pos2[:, 1].reshape(-1, SC_W),
        dst,
    )
    return _weighted_combine(ya, yb, expert_p[:, 0:1], expert_p[:, 1:2])


# speedup vs baseline: 1.0263x; 1.0263x over previous
import functools

import jax
import jax.numpy as jnp
from jax.experimental import pallas as pl
from jax.experimental.pallas import tpu as pltpu
from jax.experimental.pallas import tpu_sc as plsc

E = 8
TOPK = 2
BM = 256
SC_W = 128
SC_C = 64

_VECTOR_MESH = plsc.VectorSubcoreMesh(
    core_axis_name="core", subcore_axis_name="subcore")


def _routing(expert_idxs):
    S = expert_idxs.shape[0] * expert_idxs.shape[1]
    NB = S // BM + E
    flat_e = expert_idxs.reshape(S).astype(jnp.int32)
    onehot = (flat_e[:, None] == jnp.arange(E, dtype=jnp.int32)[None, :])
    onehot_i = onehot.astype(jnp.int32)
    ranks = jnp.cumsum(onehot_i, axis=0) - onehot_i
    counts = jnp.sum(onehot_i, axis=0)
    padded = ((counts + BM - 1) // BM) * BM
    pad_off = jnp.cumsum(padded) - padded
    pos = jnp.sum(jnp.where(onehot, ranks + pad_off[None, :], 0), axis=1)
    pos = pos.astype(jnp.int32)
    bstart = (jnp.arange(NB, dtype=jnp.int32) * BM)[:, None]
    block_expert = jnp.sum((bstart >= pad_off[None, :]).astype(jnp.int32),
                           axis=1) - 1
    block_valid = (bstart[:, 0] < jnp.sum(padded)).astype(jnp.int32)
    return pos, block_expert.astype(jnp.int32), block_valid


@functools.partial(jax.jit, static_argnames=("n_rows",))
def _sc_dispatch(x, tok_idx, pos_a, pos_b, n_rows):
    nw = tok_idx.shape[0]
    D = x.shape[1]

    @functools.partial(
        pl.kernel,
        out_type=jax.ShapeDtypeStruct((n_rows, D), x.dtype),
        mesh=_VECTOR_MESH,
        scratch_types=[pltpu.VMEM((SC_C, D), x.dtype)],
    )
    def run(x_hbm, tok_hbm, pa_hbm, pb_hbm, o_hbm, buf):
        def body(tok_vmem, pa_vmem, pb_vmem):
            for c in range(0, SC_W, SC_C):
                pltpu.sync_copy(x_hbm.at[tok_vmem.at[0, pl.ds(c, SC_C)]], buf)
                pltpu.sync_copy(buf, o_hbm.at[pa_vmem.at[0, pl.ds(c, SC_C)]])
                pltpu.sync_copy(buf, o_hbm.at[pb_vmem.at[0, pl.ds(c, SC_C)]])

        pltpu.emit_pipeline(
            body,
            grid=(nw,),
            in_specs=[
                pl.BlockSpec((1, SC_W), index_map=lambda i: (i, 0)),
                pl.BlockSpec((1, SC_W), index_map=lambda i: (i, 0)),
                pl.BlockSpec((1, SC_W), index_map=lambda i: (i, 0)),
            ],
            out_specs=[],
            core_axis_name=("core", "subcore"),
            dimension_semantics=(pltpu.PARALLEL,),
        )(tok_hbm, pa_hbm, pb_hbm)

    return run(x, tok_idx, pos_a, pos_b)


@jax.jit
def _sc_combine_gather(y_sorted, pos_a, pos_b, dst):
    nw, _ = pos_a.shape
    T = nw * SC_W
    D = y_sorted.shape[1]
    otype = jax.ShapeDtypeStruct((T, D), y_sorted.dtype)

    @functools.partial(
        pl.kernel, out_type=(otype, otype), mesh=_VECTOR_MESH,
        scratch_types=[pltpu.VMEM((SC_C, D), y_sorted.dtype)],
    )
    def run(ys_hbm, pa_hbm, pb_hbm, dst_hbm, oa_hbm, ob_hbm, buf):
        def body(pa_vmem, pb_vmem, dst_vmem):
            for c in range(0, SC_W, SC_C):
                dsts = dst_vmem.at[0, pl.ds(c, SC_C)]
                pltpu.sync_copy(ys_hbm.at[pa_vmem.at[0, pl.ds(c, SC_C)]], buf)
                pltpu.sync_copy(buf, oa_hbm.at[dsts])
                pltpu.sync_copy(ys_hbm.at[pb_vmem.at[0, pl.ds(c, SC_C)]], buf)
                pltpu.sync_copy(buf, ob_hbm.at[dsts])

        pltpu.emit_pipeline(
            body,
            grid=(nw,),
            in_specs=[
                pl.BlockSpec((1, SC_W), index_map=lambda i: (i, 0)),
                pl.BlockSpec((1, SC_W), index_map=lambda i: (i, 0)),
                pl.BlockSpec((1, SC_W), index_map=lambda i: (i, 0)),
            ],
            out_specs=[],
            core_axis_name=("core", "subcore"),
            dimension_semantics=(pltpu.PARALLEL,),
        )(pa_hbm, pb_hbm, dst_hbm)

    return run(y_sorted, pos_a, pos_b, dst)


def _ffn_block(be_ref, bv_ref, x_ref, w1_ref, w2_ref, o_ref):
    @pl.when(bv_ref[pl.program_id(0)] != 0)
    def _():
        h = jnp.dot(x_ref[...].astype(jnp.bfloat16), w1_ref[0],
                    preferred_element_type=jnp.float32)
        h = jax.nn.gelu(h.astype(jnp.bfloat16))
        o_ref[...] = jnp.dot(h, w2_ref[0].astype(jnp.bfloat16),
                             preferred_element_type=jnp.float32)


@functools.partial(jax.jit, static_argnames=("nb",))
def _grouped_ffn(xin, W1b, W2, block_expert, block_valid, nb):
    D = xin.shape[1]
    H = W1b.shape[2]
    grid_spec = pltpu.PrefetchScalarGridSpec(
        num_scalar_prefetch=2,
        grid=(nb,),
        in_specs=[
            pl.BlockSpec((BM, D), lambda b, be, bv: (b, 0)),
            pl.BlockSpec((1, D, H), lambda b, be, bv: (be[b], 0, 0)),
            pl.BlockSpec((1, H, D), lambda b, be, bv: (be[b], 0, 0)),
        ],
        out_specs=pl.BlockSpec((BM, D), lambda b, be, bv: (b, 0)),
    )
    return pl.pallas_call(
        _ffn_block,
        grid_spec=grid_spec,
        out_shape=jax.ShapeDtypeStruct((nb * BM, D), jnp.float32),
    )(block_expert, block_valid, xin, W1b, W2)


def _combine_block(ya_ref, yb_ref, pa_ref, pb_ref, o_ref):
    o_ref[...] = pa_ref[...] * ya_ref[...] + pb_ref[...] * yb_ref[...]


@jax.jit
def _weighted_combine(ya, yb, pa, pb):
    T, D = ya.shape
    BT = 512
    return pl.pallas_call(
        _combine_block,
        grid=(T // BT,),
        in_specs=[
            pl.BlockSpec((BT, D), lambda i: (i, 0)),
            pl.BlockSpec((BT, D), lambda i: (i, 0)),
            pl.BlockSpec((BT, 1), lambda i: (i, 0)),
            pl.BlockSpec((BT, 1), lambda i: (i, 0)),
        ],
        out_specs=pl.BlockSpec((BT, D), lambda i: (i, 0)),
        out_shape=jax.ShapeDtypeStruct((T, D), jnp.float32),
    )(ya, yb, pa, pb)


def kernel(x, expert_p, expert_idxs, W1, W2):
    T, D = x.shape
    S = T * TOPK
    NB = S // BM + E

    pos, block_expert, block_valid = _routing(expert_idxs)


    pos2 = pos.reshape(T, TOPK)
    pos_a = pos2[:, 0].reshape(-1, SC_W)
    pos_b = pos2[:, 1].reshape(-1, SC_W)
    tok_idx = jnp.arange(T, dtype=jnp.int32).reshape(-1, SC_W)
    xin = _sc_dispatch(x, tok_idx, pos_a, pos_b, NB * BM)

    W1b = W1.astype(jnp.bfloat16)
    y_sorted = _grouped_ffn(xin, W1b, W2, block_expert, block_valid, NB)

    ya, yb = _sc_combine_gather(y_sorted, pos_a, pos_b, tok_idx)
    return _weighted_combine(ya, yb, expert_p[:, 0:1], expert_p[:, 1:2])
